# baseline (speedup 1.0000x reference)
import functools

import numpy as np
import jax
import jax.numpy as jnp
from jax import lax
from jax.experimental import pallas as pl
from jax.experimental.pallas import tpu as pltpu
from jax.experimental.pallas import tpu_sc as plsc

NUM_CORES = 2
NUM_SUBCORES = 16
NUM_WORKERS = NUM_CORES * NUM_SUBCORES


def _pick_chunk(flat_per_worker: int) -> int:
    for c in (800, 400, 200, 80, 40, 16, 8):
        if flat_per_worker % c == 0:
            return c
    return flat_per_worker


def _make_sc_pooled_sum(batch, seq, vocab, dim):
    flat_per_worker = (batch * seq) // NUM_WORKERS
    b_per_w = batch // NUM_WORKERS
    chunk = _pick_chunk(flat_per_worker)
    n_chunks = flat_per_worker // chunk

    mesh = plsc.VectorSubcoreMesh(core_axis_name="c", subcore_axis_name="s")

    @functools.partial(
        pl.kernel,
        out_type=jax.ShapeDtypeStruct((batch, dim), jnp.float32),
        mesh=mesh,
        scratch_types=[
            pltpu.VMEM((chunk,), jnp.int32),
            pltpu.VMEM((chunk,), jnp.int32),
            pltpu.VMEM((chunk, dim), jnp.float32),
            pltpu.VMEM_SHARED((batch, dim), jnp.float32),
            pltpu.SemaphoreType.DMA,
        ],
        compiler_params=pltpu.CompilerParams(use_tc_tiling_on_sc=False),
    )
    def sc_pooled_sum(table_hbm, xflat_hbm, dest_hbm, zeros_hbm, out_hbm,
                      idx_v, dest_v, rows_v, acc_sh, sem):
        wid = lax.axis_index("s") * NUM_CORES + lax.axis_index("c")
        row_base = wid * b_per_w
        flat_base = wid * flat_per_worker

        pltpu.sync_copy(zeros_hbm, acc_sh.at[pl.ds(row_base, b_per_w)])

        @pl.loop(0, n_chunks)
        def _(i):
            off = flat_base + i * chunk
            pltpu.sync_copy(xflat_hbm.at[pl.ds(off, chunk)], idx_v)
            pltpu.sync_copy(dest_hbm.at[pl.ds(off, chunk)], dest_v)
            pltpu.async_copy(table_hbm.at[idx_v], rows_v, sem).wait()
            pltpu.sync_copy(rows_v, acc_sh.at[dest_v], add=True)

        pltpu.sync_copy(acc_sh.at[pl.ds(row_base, b_per_w)],
                        out_hbm.at[pl.ds(row_base, b_per_w)])

    return sc_pooled_sum


def _tc_retile(emb_t, col_block=512):
    dim, vocab = emb_t.shape
    grid = pl.cdiv(vocab, col_block)

    def body(in_ref, out_ref, scr):
        scr[...] = in_ref[...].T
        out_ref[:, 0:dim] = scr[pl.Slice(0, col_block // 2, 2), :]
        out_ref[:, dim:2 * dim] = scr[pl.Slice(1, col_block // 2, 2), :]

    return pl.pallas_call(
        body,
        grid=(grid,),
        in_specs=[pl.BlockSpec((dim, col_block), lambda i: (0, i))],
        out_specs=pl.BlockSpec((col_block // 2, 2 * dim), lambda i: (i, 0)),
        out_shape=jax.ShapeDtypeStruct((vocab // 2, 2 * dim), jnp.float32),
        scratch_shapes=[pltpu.VMEM((col_block, dim), jnp.float32)],
    )(emb_t)


def _tc_linear(pooled_sum, w_scaled, bias):
    batch, dim = pooled_sum.shape
    out_dim = w_scaled.shape[1]

    def body(p_ref, w_ref, b_ref, o_ref):
        o_ref[...] = (
            jnp.dot(p_ref[...], w_ref[...], preferred_element_type=jnp.float32)
            + b_ref[...]
        )

    return pl.pallas_call(
        body,
        out_shape=jax.ShapeDtypeStruct((batch, out_dim), jnp.float32),
    )(pooled_sum, w_scaled, bias)


@jax.jit
def kernel(x, emb_table, fc_w, fc_b):
    batch, seq = x.shape
    vocab, dim = emb_table.shape

    table_lin = _tc_retile(emb_table.T).reshape(vocab, dim)
    xflat = x.reshape(-1).astype(jnp.int32)
    dest = jnp.asarray(np.repeat(np.arange(batch, dtype=np.int32), seq))
    zeros = jnp.asarray(np.zeros((batch // NUM_WORKERS, dim), np.float32))

    sc_fn = _make_sc_pooled_sum(batch, seq, vocab, dim)
    pooled_sum = sc_fn(table_lin, xflat, dest, zeros)

    w_scaled = fc_w.T * (1.0 / seq)
    bias = fc_b.reshape(1, -1)
    return _tc_linear(pooled_sum, w_scaled, bias)


# speedup vs baseline: 1.0089x; 1.0089x over previous
import functools

import numpy as np
import jax
import jax.numpy as jnp
from jax import lax
from jax.experimental import pallas as pl
from jax.experimental.pallas import tpu as pltpu
from jax.experimental.pallas import tpu_sc as plsc

NUM_CORES = 2
NUM_SUBCORES = 16
NUM_WORKERS = NUM_CORES * NUM_SUBCORES


def _pick_chunk(flat_per_worker: int) -> int:
    for c in (800, 400, 200, 80, 40, 16, 8):
        if flat_per_worker % c == 0:
            return c
    return flat_per_worker


def _make_sc_pooled_sum(batch, seq, vocab, dim):
    flat_per_worker = (batch * seq) // NUM_WORKERS
    b_per_w = batch // NUM_WORKERS
    chunk = _pick_chunk(flat_per_worker)
    n_chunks = flat_per_worker // chunk

    mesh = plsc.VectorSubcoreMesh(core_axis_name="c", subcore_axis_name="s")

    @functools.partial(
        pl.kernel,
        out_type=jax.ShapeDtypeStruct((batch, dim), jnp.float32),
        mesh=mesh,
        scratch_types=[
            pltpu.VMEM((chunk,), jnp.int32),
            pltpu.VMEM((chunk,), jnp.int32),
            pltpu.VMEM((chunk, dim), jnp.float32),
            pltpu.VMEM_SHARED((batch, dim), jnp.float32),
            pltpu.SemaphoreType.DMA,
        ],
        compiler_params=pltpu.CompilerParams(use_tc_tiling_on_sc=False),
    )
    def sc_pooled_sum(table_hbm, xflat_hbm, dest_hbm, zeros_hbm, out_hbm,
                      idx_v, dest_v, rows_v, acc_sh, sem):
        wid = lax.axis_index("s") * NUM_CORES + lax.axis_index("c")
        row_base = wid * b_per_w
        flat_base = wid * flat_per_worker

        pltpu.sync_copy(zeros_hbm, acc_sh.at[pl.ds(row_base, b_per_w)])

        @pl.loop(0, n_chunks)
        def _(i):
            off = flat_base + i * chunk
            pltpu.sync_copy(xflat_hbm.at[pl.ds(off, chunk)], idx_v)
            pltpu.sync_copy(dest_hbm.at[pl.ds(off, chunk)], dest_v)
            pltpu.async_copy(table_hbm.at[idx_v], rows_v, sem).wait()
            pltpu.sync_copy(rows_v, acc_sh.at[dest_v], add=True)

        pltpu.sync_copy(acc_sh.at[pl.ds(row_base, b_per_w)],
                        out_hbm.at[pl.ds(row_base, b_per_w)])

    return sc_pooled_sum


RETILE_COLS = 512


def _tc_retile(emb_t):
    dim, vocab = emb_t.shape
    half = RETILE_COLS // 2
    grid = pl.cdiv(vocab, RETILE_COLS)
    out_rows = grid * half

    def body(in_ref, out_ref):
        out_ref[:, 0:dim] = in_ref[:, 0:half].T
        out_ref[:, dim:2 * dim] = in_ref[:, half:RETILE_COLS].T

    return pl.pallas_call(
        body,
        grid=(grid,),
        in_specs=[pl.BlockSpec((dim, RETILE_COLS), lambda i: (0, i))],
        out_specs=pl.BlockSpec((half, 2 * dim), lambda i: (i, 0)),
        out_shape=jax.ShapeDtypeStruct((out_rows, 2 * dim), jnp.float32),
    )(emb_t)


def _gather_index(x):
    c, h = RETILE_COLS, RETILE_COLS // 2
    return (x // c) * c + (x % h) * 2 + (x // h) % 2


def _tc_linear(pooled_sum, w_scaled, bias):
    batch, dim = pooled_sum.shape
    out_dim = w_scaled.shape[1]

    def body(p_ref, w_ref, b_ref, o_ref):
        o_ref[...] = (
            jnp.dot(p_ref[...], w_ref[...], preferred_element_type=jnp.float32)
            + b_ref[...]
        )

    return pl.pallas_call(
        body,
        out_shape=jax.ShapeDtypeStruct((batch, out_dim), jnp.float32),
    )(pooled_sum, w_scaled, bias)


@jax.jit
def kernel(x, emb_table, fc_w, fc_b):
    batch, seq = x.shape
    vocab, dim = emb_table.shape

    table2 = _tc_retile(emb_table.T)
    table_lin = table2.reshape(table2.shape[0] * 2, dim)
    xflat = _gather_index(x.reshape(-1).astype(jnp.int32))
    dest = jnp.asarray(np.repeat(np.arange(batch, dtype=np.int32), seq))
    zeros = jnp.asarray(np.zeros((batch // NUM_WORKERS, dim), np.float32))

    sc_fn = _make_sc_pooled_sum(batch, seq, table_lin.shape[0], dim)
    pooled_sum = sc_fn(table_lin, xflat, dest, zeros)

    w_scaled = fc_w.T * (1.0 / seq)
    bias = fc_b.reshape(1, -1)
    return _tc_linear(pooled_sum, w_scaled, bias)


# speedup vs baseline: 2.0617x; 2.0435x over previous
import functools

import numpy as np
import jax
import jax.numpy as jnp
from jax import lax
from jax.experimental import pallas as pl
from jax.experimental.pallas import tpu as pltpu
from jax.experimental.pallas import tpu_sc as plsc

NUM_CORES = 2
NUM_SUBCORES = 16
NUM_WORKERS = NUM_CORES * NUM_SUBCORES


def _pick_chunk(flat_per_worker: int) -> int:
    for c in (800, 400, 200, 80, 40, 16, 8):
        if flat_per_worker % c == 0:
            return c
    return flat_per_worker


def _make_sc_pooled_sum(batch, seq, vocab, dim):
    flat_per_worker = (batch * seq) // NUM_WORKERS
    b_per_w = batch // NUM_WORKERS
    chunk = _pick_chunk(flat_per_worker)
    n_chunks = flat_per_worker // chunk

    mesh = plsc.VectorSubcoreMesh(core_axis_name="c", subcore_axis_name="s")

    @functools.partial(
        pl.kernel,
        out_type=jax.ShapeDtypeStruct((batch, dim), jnp.float32),
        mesh=mesh,
        scratch_types=[
            pltpu.VMEM((chunk,), jnp.int32),
            pltpu.VMEM((chunk,), jnp.int32),
            pltpu.VMEM((chunk, dim), jnp.float32),
            pltpu.VMEM_SHARED((batch, dim), jnp.float32),
            pltpu.SemaphoreType.DMA,
        ],
        compiler_params=pltpu.CompilerParams(use_tc_tiling_on_sc=False),
    )
    def sc_pooled_sum(table_hbm, xflat_hbm, dest_hbm, zeros_hbm, out_hbm,
                      idx_v, dest_v, rows_v, acc_sh, sem):
        wid = lax.axis_index("s") * NUM_CORES + lax.axis_index("c")
        row_base = wid * b_per_w
        flat_base = wid * flat_per_worker

        pltpu.sync_copy(zeros_hbm, acc_sh.at[pl.ds(row_base, b_per_w)])

        @pl.loop(0, n_chunks)
        def _(i):
            off = flat_base + i * chunk
            pltpu.sync_copy(xflat_hbm.at[pl.ds(off, chunk)], idx_v)
            pltpu.sync_copy(dest_hbm.at[pl.ds(off, chunk)], dest_v)
            pltpu.async_copy(table_hbm.at[idx_v], rows_v, sem).wait()
            pltpu.sync_copy(rows_v, acc_sh.at[dest_v], add=True)

        pltpu.sync_copy(acc_sh.at[pl.ds(row_base, b_per_w)],
                        out_hbm.at[pl.ds(row_base, b_per_w)])

    return sc_pooled_sum


RETILE_COLS = 2048


def _tc_retile(emb_t):
    dim, vocab = emb_t.shape
    half = RETILE_COLS // 2
    grid = pl.cdiv(vocab, RETILE_COLS)
    out_rows = grid * half

    def body(in_ref, out_ref):
        out_ref[:, 0:dim] = in_ref[:, 0:half].T
        out_ref[:, dim:2 * dim] = in_ref[:, half:RETILE_COLS].T

    return pl.pallas_call(
        body,
        grid=(grid,),
        in_specs=[pl.BlockSpec((dim, RETILE_COLS), lambda i: (0, i))],
        out_specs=pl.BlockSpec((half, 2 * dim), lambda i: (i, 0)),
        out_shape=jax.ShapeDtypeStruct((out_rows, 2 * dim), jnp.float32),
    )(emb_t)


def _gather_index(x):
    c, h = RETILE_COLS, RETILE_COLS // 2
    return (x // c) * c + (x % h) * 2 + (x // h) % 2


def _tc_linear(pooled_sum, w_scaled, bias):
    batch, dim = pooled_sum.shape
    out_dim = w_scaled.shape[1]

    def body(p_ref, w_ref, b_ref, o_ref):
        o_ref[...] = (
            jnp.dot(p_ref[...], w_ref[...], preferred_element_type=jnp.float32)
            + b_ref[...]
        )

    return pl.pallas_call(
        body,
        out_shape=jax.ShapeDtypeStruct((batch, out_dim), jnp.float32),
    )(pooled_sum, w_scaled, bias)


@jax.jit
def kernel(x, emb_table, fc_w, fc_b):
    batch, seq = x.shape
    vocab, dim = emb_table.shape

    table2 = _tc_retile(emb_table.T)
    table_lin = table2.reshape(table2.shape[0] * 2, dim)
    xflat = _gather_index(x.reshape(-1).astype(jnp.int32))
    dest = jnp.asarray(np.repeat(np.arange(batch, dtype=np.int32), seq))
    zeros = jnp.asarray(np.zeros((batch // NUM_WORKERS, dim), np.float32))

    sc_fn = _make_sc_pooled_sum(batch, seq, table_lin.shape[0], dim)
    pooled_sum = sc_fn(table_lin, xflat, dest, zeros)

    w_scaled = fc_w.T * (1.0 / seq)
    bias = fc_b.reshape(1, -1)
    return _tc_linear(pooled_sum, w_scaled, bias)


# speedup vs baseline: 2.7871x; 1.3519x over previous
import functools

import numpy as np
import jax
import jax.numpy as jnp
from jax import lax
from jax.experimental import pallas as pl
from jax.experimental.pallas import tpu as pltpu
from jax.experimental.pallas import tpu_sc as plsc

NUM_CORES = 2
NUM_SUBCORES = 16
NUM_WORKERS = NUM_CORES * NUM_SUBCORES


def _pick_chunk(flat_per_worker: int) -> int:
    for c in (800, 400, 200, 80, 40, 16, 8):
        if flat_per_worker % c == 0:
            return c
    return flat_per_worker


def _make_sc_pooled_sum(batch, seq, vocab, dim):
    flat_per_worker = (batch * seq) // NUM_WORKERS
    b_per_w = batch // NUM_WORKERS
    chunk = _pick_chunk(flat_per_worker)
    n_chunks = flat_per_worker // chunk

    mesh = plsc.VectorSubcoreMesh(core_axis_name="c", subcore_axis_name="s")

    @functools.partial(
        pl.kernel,
        out_type=jax.ShapeDtypeStruct((batch, dim), jnp.float32),
        mesh=mesh,
        scratch_types=[
            pltpu.VMEM((chunk,), jnp.int32),
            pltpu.VMEM((chunk,), jnp.int32),
            pltpu.VMEM((chunk, dim), jnp.float32),
            pltpu.VMEM_SHARED((batch, dim), jnp.float32),
            pltpu.SemaphoreType.DMA,
        ],
        compiler_params=pltpu.CompilerParams(use_tc_tiling_on_sc=False),
    )
    def sc_pooled_sum(table_hbm, xflat_hbm, dest_hbm, zeros_hbm, out_hbm,
                      idx_v, dest_v, rows_v, acc_sh, sem):
        wid = lax.axis_index("s") * NUM_CORES + lax.axis_index("c")
        row_base = wid * b_per_w
        flat_base = wid * flat_per_worker

        pltpu.sync_copy(zeros_hbm, acc_sh.at[pl.ds(row_base, b_per_w)])

        @pl.loop(0, n_chunks)
        def _(i):
            off = flat_base + i * chunk
            pltpu.sync_copy(xflat_hbm.at[pl.ds(off, chunk)], idx_v)
            pltpu.sync_copy(dest_hbm.at[pl.ds(off, chunk)], dest_v)
            pltpu.async_copy(table_hbm.at[idx_v], rows_v, sem).wait()
            pltpu.sync_copy(rows_v, acc_sh.at[dest_v], add=True)

        pltpu.sync_copy(acc_sh.at[pl.ds(row_base, b_per_w)],
                        out_hbm.at[pl.ds(row_base, b_per_w)])

    return sc_pooled_sum


RETILE_COLS = 8192


def _tc_retile(emb_t):
    dim, vocab = emb_t.shape
    half = RETILE_COLS // 2
    grid = pl.cdiv(vocab, RETILE_COLS)
    out_rows = grid * half

    def body(in_ref, out_ref):
        out_ref[:, 0:dim] = in_ref[:, 0:half].T
        out_ref[:, dim:2 * dim] = in_ref[:, half:RETILE_COLS].T

    return pl.pallas_call(
        body,
        grid=(grid,),
        in_specs=[pl.BlockSpec((dim, RETILE_COLS), lambda i: (0, i))],
        out_specs=pl.BlockSpec((half, 2 * dim), lambda i: (i, 0)),
        out_shape=jax.ShapeDtypeStruct((out_rows, 2 * dim), jnp.float32),
    )(emb_t)


def _gather_index(x):
    c, h = RETILE_COLS, RETILE_COLS // 2
    return (x // c) * c + (x % h) * 2 + (x // h) % 2


def _tc_linear(pooled_sum, w_scaled, bias):
    batch, dim = pooled_sum.shape
    out_dim = w_scaled.shape[1]

    def body(p_ref, w_ref, b_ref, o_ref):
        o_ref[...] = (
            jnp.dot(p_ref[...], w_ref[...], preferred_element_type=jnp.float32)
            + b_ref[...]
        )

    return pl.pallas_call(
        body,
        out_shape=jax.ShapeDtypeStruct((batch, out_dim), jnp.float32),
    )(pooled_sum, w_scaled, bias)


@jax.jit
def kernel(x, emb_table, fc_w, fc_b):
    batch, seq = x.shape
    vocab, dim = emb_table.shape

    table2 = _tc_retile(emb_table.T)
    table_lin = table2.reshape(table2.shape[0] * 2, dim)
    xflat = _gather_index(x.reshape(-1).astype(jnp.int32))
    dest = jnp.asarray(np.repeat(np.arange(batch, dtype=np.int32), seq))
    zeros = jnp.asarray(np.zeros((batch // NUM_WORKERS, dim), np.float32))

    sc_fn = _make_sc_pooled_sum(batch, seq, table_lin.shape[0], dim)
    pooled_sum = sc_fn(table_lin, xflat, dest, zeros)

    w_scaled = fc_w.T * (1.0 / seq)
    bias = fc_b.reshape(1, -1)
    return _tc_linear(pooled_sum, w_scaled, bias)


# speedup vs baseline: 2.9628x; 1.0630x over previous
import functools

import numpy as np
import jax
import jax.numpy as jnp
from jax import lax
from jax.experimental import pallas as pl
from jax.experimental.pallas import tpu as pltpu
from jax.experimental.pallas import tpu_sc as plsc

NUM_CORES = 2
NUM_SUBCORES = 16
NUM_WORKERS = NUM_CORES * NUM_SUBCORES


def _pick_chunk(flat_per_worker: int) -> int:
    for c in (800, 400, 200, 80, 40, 16, 8):
        if flat_per_worker % c == 0:
            return c
    return flat_per_worker


def _make_sc_pooled_sum(batch, seq, vocab, dim):
    flat_per_worker = (batch * seq) // NUM_WORKERS
    b_per_w = batch // NUM_WORKERS
    chunk = _pick_chunk(flat_per_worker)
    n_chunks = flat_per_worker // chunk

    mesh = plsc.VectorSubcoreMesh(core_axis_name="c", subcore_axis_name="s")

    @functools.partial(
        pl.kernel,
        out_type=jax.ShapeDtypeStruct((batch, dim), jnp.float32),
        mesh=mesh,
        scratch_types=[
            pltpu.VMEM((chunk,), jnp.int32),
            pltpu.VMEM((chunk,), jnp.int32),
            pltpu.VMEM((chunk, dim), jnp.float32),
            pltpu.VMEM_SHARED((batch, dim), jnp.float32),
            pltpu.SemaphoreType.DMA,
        ],
        compiler_params=pltpu.CompilerParams(use_tc_tiling_on_sc=False),
    )
    def sc_pooled_sum(table_hbm, xflat_hbm, dest_hbm, zeros_hbm, out_hbm,
                      idx_v, dest_v, rows_v, acc_sh, sem):
        wid = lax.axis_index("s") * NUM_CORES + lax.axis_index("c")
        row_base = wid * b_per_w
        flat_base = wid * flat_per_worker

        pltpu.sync_copy(zeros_hbm, acc_sh.at[pl.ds(row_base, b_per_w)])

        @pl.loop(0, n_chunks)
        def _(i):
            off = flat_base + i * chunk
            pltpu.sync_copy(xflat_hbm.at[pl.ds(off, chunk)], idx_v)
            pltpu.sync_copy(dest_hbm.at[pl.ds(off, chunk)], dest_v)
            pltpu.async_copy(table_hbm.at[idx_v], rows_v, sem).wait()
            pltpu.sync_copy(rows_v, acc_sh.at[dest_v], add=True)

        pltpu.sync_copy(acc_sh.at[pl.ds(row_base, b_per_w)],
                        out_hbm.at[pl.ds(row_base, b_per_w)])

    return sc_pooled_sum


RETILE_COLS = 16384


def _tc_retile(emb_t):
    dim, vocab = emb_t.shape
    half = RETILE_COLS // 2
    grid = pl.cdiv(vocab, RETILE_COLS)
    out_rows = grid * half

    def body(in_ref, out_ref):
        out_ref[:, 0:dim] = in_ref[:, 0:half].T
        out_ref[:, dim:2 * dim] = in_ref[:, half:RETILE_COLS].T

    return pl.pallas_call(
        body,
        grid=(grid,),
        in_specs=[pl.BlockSpec((dim, RETILE_COLS), lambda i: (0, i))],
        out_specs=pl.BlockSpec((half, 2 * dim), lambda i: (i, 0)),
        out_shape=jax.ShapeDtypeStruct((out_rows, 2 * dim), jnp.float32),
    )(emb_t)


def _gather_index(x):
    c, h = RETILE_COLS, RETILE_COLS // 2
    return (x // c) * c + (x % h) * 2 + (x // h) % 2


def _tc_linear(pooled_sum, w_scaled, bias):
    batch, dim = pooled_sum.shape
    out_dim = w_scaled.shape[1]

    def body(p_ref, w_ref, b_ref, o_ref):
        o_ref[...] = (
            jnp.dot(p_ref[...], w_ref[...], preferred_element_type=jnp.float32)
            + b_ref[...]
        )

    return pl.pallas_call(
        body,
        out_shape=jax.ShapeDtypeStruct((batch, out_dim), jnp.float32),
    )(pooled_sum, w_scaled, bias)


@jax.jit
def kernel(x, emb_table, fc_w, fc_b):
    batch, seq = x.shape
    vocab, dim = emb_table.shape

    table2 = _tc_retile(emb_table.T)
    table_lin = table2.reshape(table2.shape[0] * 2, dim)
    xflat = _gather_index(x.reshape(-1).astype(jnp.int32))
    dest = jnp.asarray(np.repeat(np.arange(batch, dtype=np.int32), seq))
    zeros = jnp.asarray(np.zeros((batch // NUM_WORKERS, dim), np.float32))

    sc_fn = _make_sc_pooled_sum(batch, seq, table_lin.shape[0], dim)
    pooled_sum = sc_fn(table_lin, xflat, dest, zeros)

    w_scaled = fc_w.T * (1.0 / seq)
    bias = fc_b.reshape(1, -1)
    return _tc_linear(pooled_sum, w_scaled, bias)


# speedup vs baseline: 3.0500x; 1.0294x over previous
import functools

import numpy as np
import jax
import jax.numpy as jnp
from jax import lax
from jax.experimental import pallas as pl
from jax.experimental.pallas import tpu as pltpu
from jax.experimental.pallas import tpu_sc as plsc

NUM_CORES = 2
NUM_SUBCORES = 16
NUM_WORKERS = NUM_CORES * NUM_SUBCORES


def _pick_chunk(flat_per_worker: int) -> int:
    for c in (800, 400, 200, 80, 40, 16, 8):
        if flat_per_worker % c == 0:
            return c
    return flat_per_worker


def _make_sc_pooled_sum(batch, seq, vocab, dim):
    flat_per_worker = (batch * seq) // NUM_WORKERS
    b_per_w = batch // NUM_WORKERS
    chunk = _pick_chunk(flat_per_worker)
    n_chunks = flat_per_worker // chunk

    mesh = plsc.VectorSubcoreMesh(core_axis_name="c", subcore_axis_name="s")

    @functools.partial(
        pl.kernel,
        out_type=jax.ShapeDtypeStruct((batch, dim), jnp.float32),
        mesh=mesh,
        scratch_types=[
            pltpu.VMEM((chunk,), jnp.int32),
            pltpu.VMEM((chunk,), jnp.int32),
            pltpu.VMEM((chunk, dim), jnp.float32),
            pltpu.VMEM_SHARED((batch, dim), jnp.float32),
            pltpu.SemaphoreType.DMA,
        ],
        compiler_params=pltpu.CompilerParams(use_tc_tiling_on_sc=False),
    )
    def sc_pooled_sum(table_hbm, xflat_hbm, dest_hbm, zeros_hbm, out_hbm,
                      idx_v, dest_v, rows_v, acc_sh, sem):
        wid = lax.axis_index("s") * NUM_CORES + lax.axis_index("c")
        row_base = wid * b_per_w
        flat_base = wid * flat_per_worker

        pltpu.sync_copy(zeros_hbm, acc_sh.at[pl.ds(row_base, b_per_w)])

        @pl.loop(0, n_chunks)
        def _(i):
            off = flat_base + i * chunk
            pltpu.sync_copy(xflat_hbm.at[pl.ds(off, chunk)], idx_v)
            pltpu.sync_copy(dest_hbm.at[pl.ds(off, chunk)], dest_v)
            pltpu.async_copy(table_hbm.at[idx_v], rows_v, sem).wait()
            pltpu.sync_copy(rows_v, acc_sh.at[dest_v], add=True)

        pltpu.sync_copy(acc_sh.at[pl.ds(row_base, b_per_w)],
                        out_hbm.at[pl.ds(row_base, b_per_w)])

    return sc_pooled_sum


RETILE_COLS = 32768


def _tc_retile(emb_t):
    dim, vocab = emb_t.shape
    half = RETILE_COLS // 2
    grid = pl.cdiv(vocab, RETILE_COLS)
    out_rows = grid * half

    def body(in_ref, out_ref):
        out_ref[:, 0:dim] = in_ref[:, 0:half].T
        out_ref[:, dim:2 * dim] = in_ref[:, half:RETILE_COLS].T

    return pl.pallas_call(
        body,
        grid=(grid,),
        in_specs=[pl.BlockSpec((dim, RETILE_COLS), lambda i: (0, i))],
        out_specs=pl.BlockSpec((half, 2 * dim), lambda i: (i, 0)),
        out_shape=jax.ShapeDtypeStruct((out_rows, 2 * dim), jnp.float32),
    )(emb_t)


def _gather_index(x):
    c, h = RETILE_COLS, RETILE_COLS // 2
    return (x // c) * c + (x % h) * 2 + (x // h) % 2


def _tc_linear(pooled_sum, w_scaled, bias):
    batch, dim = pooled_sum.shape
    out_dim = w_scaled.shape[1]

    def body(p_ref, w_ref, b_ref, o_ref):
        o_ref[...] = (
            jnp.dot(p_ref[...], w_ref[...], preferred_element_type=jnp.float32)
            + b_ref[...]
        )

    return pl.pallas_call(
        body,
        out_shape=jax.ShapeDtypeStruct((batch, out_dim), jnp.float32),
    )(pooled_sum, w_scaled, bias)


@jax.jit
def kernel(x, emb_table, fc_w, fc_b):
    batch, seq = x.shape
    vocab, dim = emb_table.shape

    table2 = _tc_retile(emb_table.T)
    table_lin = table2.reshape(table2.shape[0] * 2, dim)
    xflat = _gather_index(x.reshape(-1).astype(jnp.int32))
    dest = jnp.asarray(np.repeat(np.arange(batch, dtype=np.int32), seq))
    zeros = jnp.asarray(np.zeros((batch // NUM_WORKERS, dim), np.float32))

    sc_fn = _make_sc_pooled_sum(batch, seq, table_lin.shape[0], dim)
    pooled_sum = sc_fn(table_lin, xflat, dest, zeros)

    w_scaled = fc_w.T * (1.0 / seq)
    bias = fc_b.reshape(1, -1)
    return _tc_linear(pooled_sum, w_scaled, bias)


# speedup vs baseline: 3.4979x; 1.1468x over previous
import functools

import numpy as np
import jax
import jax.numpy as jnp
from jax import lax
from jax.experimental import pallas as pl
from jax.experimental.pallas import tpu as pltpu
from jax.experimental.pallas import tpu_sc as plsc

NUM_CORES = 2
NUM_SUBCORES = 16
NUM_WORKERS = NUM_CORES * NUM_SUBCORES


def _pick_chunk(flat_per_worker: int) -> int:
    for c in (800, 400, 200, 80, 40, 16, 8):
        if flat_per_worker % c == 0:
            return c
    return flat_per_worker


def _make_sc_pooled_sum(batch, seq, vocab, dim):
    flat_per_worker = (batch * seq) // NUM_WORKERS
    b_per_w = batch // NUM_WORKERS
    chunk = _pick_chunk(flat_per_worker)
    n_chunks = flat_per_worker // chunk

    mesh = plsc.VectorSubcoreMesh(core_axis_name="c", subcore_axis_name="s")

    assert n_chunks % 2 == 0 and n_chunks >= 4

    @functools.partial(
        pl.kernel,
        out_type=jax.ShapeDtypeStruct((batch, dim), jnp.float32),
        mesh=mesh,
        scratch_types=[
            pltpu.VMEM((2, chunk), jnp.int32),
            pltpu.VMEM((2, chunk), jnp.int32),
            pltpu.VMEM((2, chunk, dim), jnp.float32),
            pltpu.VMEM_SHARED((batch, dim), jnp.float32),
            pltpu.SemaphoreType.DMA,
            pltpu.SemaphoreType.DMA,
            pltpu.SemaphoreType.DMA,
            pltpu.SemaphoreType.DMA,
        ],
        compiler_params=pltpu.CompilerParams(use_tc_tiling_on_sc=False),
    )
    def sc_pooled_sum(table_hbm, xflat_hbm, dest_hbm, zeros_hbm, out_hbm,
                      idx_v, dest_v, rows_v, acc_sh,
                      sem_i0, sem_i1, sem_g0, sem_g1):
        wid = lax.axis_index("s") * NUM_CORES + lax.axis_index("c")
        row_base = wid * b_per_w
        flat_base = wid * flat_per_worker
        sem_i = (sem_i0, sem_i1)
        sem_g = (sem_g0, sem_g1)

        def idx_copies(i, b):
            off = flat_base + i * chunk
            return (
                pltpu.make_async_copy(
                    xflat_hbm.at[pl.ds(off, chunk)], idx_v.at[b], sem_i[b]),
                pltpu.make_async_copy(
                    dest_hbm.at[pl.ds(off, chunk)], dest_v.at[b], sem_i[b]),
            )

        def gather(b):
            return pltpu.make_async_copy(
                table_hbm.at[idx_v.at[b]], rows_v.at[b], sem_g[b])

        for c in idx_copies(0, 0):
            c.start()
        for c in idx_copies(0, 0):
            c.wait()
        gather(0).start()
        for c in idx_copies(1, 1):
            c.start()

        pltpu.sync_copy(zeros_hbm, acc_sh.at[pl.ds(row_base, b_per_w)])

        @pl.loop(0, n_chunks, step=2)
        def _(i0):
            for b in (0, 1):
                i = i0 + b
                gather(b).wait()
                nb = 1 - b

                @pl.when(i + 1 < n_chunks)
                def _():
                    for c in idx_copies(i + 1, nb):
                        c.wait()
                    gather(nb).start()

                pltpu.sync_copy(rows_v.at[b], acc_sh.at[dest_v.at[b]],
                                add=True)

                @pl.when(i + 2 < n_chunks)
                def _():
                    for c in idx_copies(i + 2, b):
                        c.start()

        pltpu.sync_copy(acc_sh.at[pl.ds(row_base, b_per_w)],
                        out_hbm.at[pl.ds(row_base, b_per_w)])

    return sc_pooled_sum


RETILE_COLS = 32768


def _tc_retile(emb_t):
    dim, vocab = emb_t.shape
    half = RETILE_COLS // 2
    grid = pl.cdiv(vocab, RETILE_COLS)
    out_rows = grid * half

    def body(in_ref, out_ref):
        out_ref[:, 0:dim] = in_ref[:, 0:half].T
        out_ref[:, dim:2 * dim] = in_ref[:, half:RETILE_COLS].T

    return pl.pallas_call(
        body,
        grid=(grid,),
        in_specs=[pl.BlockSpec((dim, RETILE_COLS), lambda i: (0, i))],
        out_specs=pl.BlockSpec((half, 2 * dim), lambda i: (i, 0)),
        out_shape=jax.ShapeDtypeStruct((out_rows, 2 * dim), jnp.float32),
    )(emb_t)


def _gather_index(x):
    c, h = RETILE_COLS, RETILE_COLS // 2
    return (x // c) * c + (x % h) * 2 + (x // h) % 2


def _tc_linear(pooled_sum, w_scaled, bias):
    batch, dim = pooled_sum.shape
    out_dim = w_scaled.shape[1]

    def body(p_ref, w_ref, b_ref, o_ref):
        o_ref[...] = (
            jnp.dot(p_ref[...], w_ref[...], preferred_element_type=jnp.float32)
            + b_ref[...]
        )

    return pl.pallas_call(
        body,
        out_shape=jax.ShapeDtypeStruct((batch, out_dim), jnp.float32),
    )(pooled_sum, w_scaled, bias)


@jax.jit
def kernel(x, emb_table, fc_w, fc_b):
    batch, seq = x.shape
    vocab, dim = emb_table.shape

    table2 = _tc_retile(emb_table.T)
    table_lin = table2.reshape(table2.shape[0] * 2, dim)
    xflat = _gather_index(x.reshape(-1).astype(jnp.int32))
    dest = jnp.asarray(np.repeat(np.arange(batch, dtype=np.int32), seq))
    zeros = jnp.asarray(np.zeros((batch // NUM_WORKERS, dim), np.float32))

    sc_fn = _make_sc_pooled_sum(batch, seq, table_lin.shape[0], dim)
    pooled_sum = sc_fn(table_lin, xflat, dest, zeros)

    w_scaled = fc_w.T * (1.0 / seq)
    bias = fc_b.reshape(1, -1)
    return _tc_linear(pooled_sum, w_scaled, bias)


# speedup vs baseline: 3.5385x; 1.0116x over previous
import functools

import numpy as np
import jax
import jax.numpy as jnp
from jax import lax
from jax.experimental import pallas as pl
from jax.experimental.pallas import tpu as pltpu
from jax.experimental.pallas import tpu_sc as plsc

NUM_CORES = 2
NUM_SUBCORES = 16
NUM_WORKERS = NUM_CORES * NUM_SUBCORES


def _pick_chunk(flat_per_worker: int) -> int:
    for c in (800, 400, 200, 80, 40, 16, 8):
        if flat_per_worker % c == 0:
            return c
    return flat_per_worker


def _make_sc_pooled_sum(batch, seq, vocab, dim):
    flat_per_worker = (batch * seq) // NUM_WORKERS
    b_per_w = batch // NUM_WORKERS
    chunk = _pick_chunk(flat_per_worker)
    n_chunks = flat_per_worker // chunk

    mesh = plsc.VectorSubcoreMesh(core_axis_name="c", subcore_axis_name="s")

    assert n_chunks % 2 == 0 and n_chunks >= 4

    @functools.partial(
        pl.kernel,
        out_type=jax.ShapeDtypeStruct((batch, dim), jnp.float32),
        mesh=mesh,
        scratch_types=[
            pltpu.VMEM((2, chunk), jnp.int32),
            pltpu.VMEM((2, chunk), jnp.int32),
            pltpu.VMEM((2, chunk, dim), jnp.float32),
            pltpu.VMEM_SHARED((batch, dim), jnp.float32),
            pltpu.SemaphoreType.DMA,
            pltpu.SemaphoreType.DMA,
            pltpu.SemaphoreType.DMA,
            pltpu.SemaphoreType.DMA,
        ],
        compiler_params=pltpu.CompilerParams(use_tc_tiling_on_sc=False),
    )
    def sc_pooled_sum(table_hbm, xflat_hbm, dest_hbm, zeros_hbm, out_hbm,
                      idx_v, dest_v, rows_v, acc_sh,
                      sem_i0, sem_i1, sem_g0, sem_g1):
        wid = lax.axis_index("s") * NUM_CORES + lax.axis_index("c")
        row_base = wid * b_per_w
        flat_base = wid * flat_per_worker
        sem_i = (sem_i0, sem_i1)
        sem_g = (sem_g0, sem_g1)

        def idx_copies(i, b):
            off = flat_base + i * chunk
            return (
                pltpu.make_async_copy(
                    xflat_hbm.at[pl.ds(off, chunk)], idx_v.at[b], sem_i[b]),
                pltpu.make_async_copy(
                    dest_hbm.at[pl.ds(off, chunk)], dest_v.at[b], sem_i[b]),
            )

        nsplit = 4
        sub = chunk // nsplit

        def gather(b):
            return tuple(
                pltpu.make_async_copy(
                    table_hbm.at[idx_v.at[b, pl.ds(s * sub, sub)]],
                    rows_v.at[b, pl.ds(s * sub, sub)],
                    sem_g[b])
                for s in range(nsplit)
            )

        for c in idx_copies(0, 0):
            c.start()
        for c in idx_copies(0, 0):
            c.wait()
        for g in gather(0):
            g.start()
        for c in idx_copies(1, 1):
            c.start()

        pltpu.sync_copy(zeros_hbm, acc_sh.at[pl.ds(row_base, b_per_w)])

        @pl.loop(0, n_chunks, step=2)
        def _(i0):
            for b in (0, 1):
                i = i0 + b
                for g in gather(b):
                    g.wait()
                nb = 1 - b

                @pl.when(i + 1 < n_chunks)
                def _():
                    for c in idx_copies(i + 1, nb):
                        c.wait()
                    for g in gather(nb):
                        g.start()

                pltpu.sync_copy(rows_v.at[b], acc_sh.at[dest_v.at[b]],
                                add=True)

                @pl.when(i + 2 < n_chunks)
                def _():
                    for c in idx_copies(i + 2, b):
                        c.start()

        pltpu.sync_copy(acc_sh.at[pl.ds(row_base, b_per_w)],
                        out_hbm.at[pl.ds(row_base, b_per_w)])

    return sc_pooled_sum


RETILE_COLS = 32768


def _tc_retile(emb_t):
    dim, vocab = emb_t.shape
    half = RETILE_COLS // 2
    grid = pl.cdiv(vocab, RETILE_COLS)
    out_rows = grid * half

    def body(in_ref, out_ref):
        out_ref[:, 0:dim] = in_ref[:, 0:half].T
        out_ref[:, dim:2 * dim] = in_ref[:, half:RETILE_COLS].T

    return pl.pallas_call(
        body,
        grid=(grid,),
        in_specs=[pl.BlockSpec((dim, RETILE_COLS), lambda i: (0, i))],
        out_specs=pl.BlockSpec((half, 2 * dim), lambda i: (i, 0)),
        out_shape=jax.ShapeDtypeStruct((out_rows, 2 * dim), jnp.float32),
    )(emb_t)


def _gather_index(x):
    c, h = RETILE_COLS, RETILE_COLS // 2
    return (x // c) * c + (x % h) * 2 + (x // h) % 2


def _tc_linear(pooled_sum, w_scaled, bias):
    batch, dim = pooled_sum.shape
    out_dim = w_scaled.shape[1]

    def body(p_ref, w_ref, b_ref, o_ref):
        o_ref[...] = (
            jnp.dot(p_ref[...], w_ref[...], preferred_element_type=jnp.float32)
            + b_ref[...]
        )

    return pl.pallas_call(
        body,
        out_shape=jax.ShapeDtypeStruct((batch, out_dim), jnp.float32),
    )(pooled_sum, w_scaled, bias)


@jax.jit
def kernel(x, emb_table, fc_w, fc_b):
    batch, seq = x.shape
    vocab, dim = emb_table.shape

    table2 = _tc_retile(emb_table.T)
    table_lin = table2.reshape(table2.shape[0] * 2, dim)
    xflat = _gather_index(x.reshape(-1).astype(jnp.int32))
    dest = jnp.asarray(np.repeat(np.arange(batch, dtype=np.int32), seq))
    zeros = jnp.asarray(np.zeros((batch // NUM_WORKERS, dim), np.float32))

    sc_fn = _make_sc_pooled_sum(batch, seq, table_lin.shape[0], dim)
    pooled_sum = sc_fn(table_lin, xflat, dest, zeros)

    w_scaled = fc_w.T * (1.0 / seq)
    bias = fc_b.reshape(1, -1)
    return _tc_linear(pooled_sum, w_scaled, bias)
